# baseline (device time: 20048 ns/iter reference)
import jax
import jax.numpy as jnp
from jax import lax
from jax.experimental import pallas as pl
from jax.experimental.pallas import tpu as pltpu

B, H, D, BS = 8, 8, 64, 16
NB = 64
HB = H * B
HD = H * D
SCALE = D ** -0.5
NEG = -1e30


def kernel(Q, K, V, bt, lens):
    n_local_pages = K.shape[0]
    n_keys = n_local_pages * BS

    def body(q_ref, k_hbm, v_hbm, bt_ref, lens_ref, out_ref, *scratch):
        kvs = scratch[0:H]
        vvs = scratch[H:2 * H]
        copy_sems, send_buf, recv_buf, send_sem, recv_sem = scratch[2 * H:]
        my_x = lax.axis_index("x")
        my_y = lax.axis_index("y")
        my_z = lax.axis_index("z")
        peer = (my_x, my_y, 1 - my_z)

        barrier = pltpu.get_barrier_semaphore()
        pl.semaphore_signal(barrier, inc=1, device_id=peer,
                            device_id_type=pl.DeviceIdType.MESH)

        kr = k_hbm.reshape(n_keys, H, D)
        vr = v_hbm.reshape(n_keys, H, D)
        k_cps = [
            pltpu.make_async_copy(kr.at[:, h, :], kvs[h],
                                  copy_sems.at[1])
            for h in range(H)
        ]
        v_cps = [
            pltpu.make_async_copy(vr.at[:, h, :], vvs[h],
                                  copy_sems.at[2])
            for h in range(H)
        ]
        for cp in k_cps + v_cps:
            cp.start()

        def f32(x):
            return x.astype(jnp.float32)

        def iota(shape, dim):
            return lax.broadcasted_iota(jnp.int32, shape, dim)

        rowsel = f32(iota((HB, B), 0) % B == iota((HB, B), 1))
        headmask = f32(iota((HB, HD), 0) // B == iota((HB, HD), 1) // D)

        lens_col = jnp.zeros((B, 1), jnp.int32)
        for b in range(B):
            lens_col = jnp.where(iota((B, 1), 0) == b, lens_ref[b],
                                 lens_col)

        pid_row = my_z * n_local_pages + iota((1, n_local_pages), 1)
        counts = jnp.zeros((B, n_local_pages), jnp.float32)
        for j in range(NB):
            hit = (bt_ref[:, j:j + 1] == pid_row) & (lens_col > j)
            counts = counts + f32(hit)

        expand = f32(iota((n_local_pages, n_keys), 1) // BS
                     == iota((n_local_pages, n_keys), 0))
        w = lax.dot_general(
            counts, expand,
            dimension_numbers=(((1,), (0,)), ((), ())),
            preferred_element_type=jnp.float32,
        )
        wexp = lax.dot_general(
            rowsel, w,
            dimension_numbers=(((1,), (0,)), ((), ())),
            preferred_element_type=jnp.float32,
        )

        qflat = jnp.concatenate(
            [q_ref[:, 0, h, :] for h in range(H)], axis=1)
        qexp = lax.dot_general(
            rowsel, qflat,
            dimension_numbers=(((1,), (0,)), ((), ())),
            preferred_element_type=jnp.float32,
        )
        qbd = qexp * headmask

        for cp in k_cps:
            cp.wait()
        kvf = jnp.concatenate([r[...] for r in kvs], axis=1)
        S = lax.dot_general(
            qbd, kvf,
            dimension_numbers=(((1,), (1,)), ((), ())),
            preferred_element_type=jnp.float32,
        ) * SCALE

        valid = wexp > 0.0
        Sm = jnp.where(valid, S, NEG)
        m_l = jnp.max(Sm, axis=1, keepdims=True)
        p = wexp * jnp.exp(Sm - m_l)
        s_l = jnp.sum(p, axis=1, keepdims=True)

        for cp in v_cps:
            cp.wait()
        vvf = jnp.concatenate([r[...] for r in vvs], axis=1)
        A2 = lax.dot_general(
            p, vvf,
            dimension_numbers=(((1,), (0,)), ((), ())),
            preferred_element_type=jnp.float32,
        ) * headmask
        fold = f32(iota((HD, D), 0) % D == iota((HD, D), 1))
        acc_l = lax.dot_general(
            A2, fold,
            dimension_numbers=(((1,), (0,)), ((), ())),
            preferred_element_type=jnp.float32,
        )

        send_buf[0] = acc_l
        send_buf[1] = jnp.broadcast_to(m_l, (HB, D))
        send_buf[2] = jnp.broadcast_to(s_l, (HB, D))

        pl.semaphore_wait(barrier, 1)
        rdma = pltpu.make_async_remote_copy(
            src_ref=send_buf,
            dst_ref=recv_buf,
            send_sem=send_sem,
            recv_sem=recv_sem,
            device_id=peer,
            device_id_type=pl.DeviceIdType.MESH,
        )
        rdma.start()
        rdma.wait()

        acc_r = recv_buf[0]
        m_r = recv_buf[1, :, 0:1]
        s_r = recv_buf[2, :, 0:1]

        m = jnp.maximum(m_l, m_r)
        a_l = jnp.exp(m_l - m)
        a_r = jnp.exp(m_r - m)
        denom = a_l * s_l + a_r * s_r
        res = (a_l * acc_l + a_r * acc_r) / denom

        for h in range(H):
            out_ref[:, 0, h, :] = res[h * B:(h + 1) * B, :]

    return pl.pallas_call(
        body,
        out_shape=jax.ShapeDtypeStruct((B, 1, H, D), jnp.float32),
        in_specs=[
            pl.BlockSpec(memory_space=pltpu.VMEM),
            pl.BlockSpec(memory_space=pltpu.MemorySpace.HBM),
            pl.BlockSpec(memory_space=pltpu.MemorySpace.HBM),
            pl.BlockSpec(memory_space=pltpu.VMEM),
            pl.BlockSpec(memory_space=pltpu.SMEM),
        ],
        out_specs=pl.BlockSpec(memory_space=pltpu.VMEM),
        scratch_shapes=[
            *([pltpu.VMEM((n_keys, D), jnp.float32)] * (2 * H)),
            pltpu.SemaphoreType.DMA((4,)),
            pltpu.VMEM((3, HB, D), jnp.float32),
            pltpu.VMEM((3, HB, D), jnp.float32),
            pltpu.SemaphoreType.DMA,
            pltpu.SemaphoreType.DMA,
        ],
        compiler_params=pltpu.CompilerParams(collective_id=0),
    )(Q, K, V, bt, lens)


# device time: 11470 ns/iter; 1.7479x vs baseline; 1.7479x over previous
import jax
import jax.numpy as jnp
from jax import lax
from jax.experimental import pallas as pl
from jax.experimental.pallas import tpu as pltpu

B, H, D, BS = 8, 8, 64, 16
NB = 64
HB = H * B
HD = H * D
SCALE = D ** -0.5
NEG = -1e30


def kernel(Q, K, V, bt, lens):
    n_local_pages = K.shape[0]
    n_keys = n_local_pages * BS

    def body(q_ref, k_ref, v_ref, bt_ref, lens_ref, out_ref,
             send_buf, recv_buf, send_sem, recv_sem):
        my_x = lax.axis_index("x")
        my_y = lax.axis_index("y")
        my_z = lax.axis_index("z")
        peer = (my_x, my_y, 1 - my_z)

        barrier = pltpu.get_barrier_semaphore()
        pl.semaphore_signal(barrier, inc=1, device_id=peer,
                            device_id_type=pl.DeviceIdType.MESH)

        def f32(x):
            return x.astype(jnp.float32)

        def iota(shape, dim):
            return lax.broadcasted_iota(jnp.int32, shape, dim)

        rowsel = f32(iota((HB, B), 0) % B == iota((HB, B), 1))
        headmask = f32(iota((HB, HD), 0) // B == iota((HB, HD), 1) // D)

        pid_row = my_z * n_local_pages + iota((1, n_local_pages), 1)
        btT = jnp.transpose(bt_ref[...])
        j_col = iota((NB, 1), 0)
        count_rows = []
        for b in range(B):
            hit = (btT[:, b:b + 1] == pid_row) & (j_col < lens_ref[b])
            count_rows.append(jnp.sum(f32(hit), axis=0, keepdims=True))
        counts = jnp.concatenate(count_rows, axis=0)

        expand = f32(iota((n_local_pages, n_keys), 1) // BS
                     == iota((n_local_pages, n_keys), 0))
        w = lax.dot_general(
            counts, expand,
            dimension_numbers=(((1,), (0,)), ((), ())),
            preferred_element_type=jnp.float32,
        )
        wexp = lax.dot_general(
            rowsel, w,
            dimension_numbers=(((1,), (0,)), ((), ())),
            preferred_element_type=jnp.float32,
        )

        qflat = jnp.concatenate(
            [q_ref[:, 0, h, :] for h in range(H)], axis=1)
        qexp = lax.dot_general(
            rowsel, qflat,
            dimension_numbers=(((1,), (0,)), ((), ())),
            preferred_element_type=jnp.float32,
        )
        qbd = qexp * headmask

        S = lax.dot_general(
            qbd, k_ref[...],
            dimension_numbers=(((1,), (1,)), ((), ())),
            preferred_element_type=jnp.float32,
        ) * SCALE

        valid = wexp > 0.0
        Sm = jnp.where(valid, S, NEG)
        m_l = jnp.max(Sm, axis=1, keepdims=True)
        p = wexp * jnp.exp(Sm - m_l)
        s_l = jnp.sum(p, axis=1, keepdims=True)

        A2 = lax.dot_general(
            p, v_ref[...],
            dimension_numbers=(((1,), (0,)), ((), ())),
            preferred_element_type=jnp.float32,
        ) * headmask
        fold = f32(iota((HD, D), 0) % D == iota((HD, D), 1))
        acc_l = lax.dot_general(
            A2, fold,
            dimension_numbers=(((1,), (0,)), ((), ())),
            preferred_element_type=jnp.float32,
        )

        send_buf[:, 0:D] = acc_l
        send_buf[:, D:D + 1] = m_l
        send_buf[:, D + 1:D + 2] = s_l

        pl.semaphore_wait(barrier, 1)
        rdma = pltpu.make_async_remote_copy(
            src_ref=send_buf,
            dst_ref=recv_buf,
            send_sem=send_sem,
            recv_sem=recv_sem,
            device_id=peer,
            device_id_type=pl.DeviceIdType.MESH,
        )
        rdma.start()
        rdma.wait_recv()

        acc_r = recv_buf[:, 0:D]
        m_r = recv_buf[:, D:D + 1]
        s_r = recv_buf[:, D + 1:D + 2]

        m = jnp.maximum(m_l, m_r)
        a_l = jnp.exp(m_l - m)
        a_r = jnp.exp(m_r - m)
        denom = a_l * s_l + a_r * s_r
        res = (a_l * acc_l + a_r * acc_r) / denom

        for h in range(H):
            out_ref[:, 0, h, :] = res[h * B:(h + 1) * B, :]
        rdma.wait_send()

    return pl.pallas_call(
        body,
        out_shape=jax.ShapeDtypeStruct((B, 1, H, D), jnp.float32),
        in_specs=[
            pl.BlockSpec(memory_space=pltpu.VMEM),
            pl.BlockSpec(memory_space=pltpu.VMEM),
            pl.BlockSpec(memory_space=pltpu.VMEM),
            pl.BlockSpec(memory_space=pltpu.VMEM),
            pl.BlockSpec(memory_space=pltpu.SMEM),
        ],
        out_specs=pl.BlockSpec(memory_space=pltpu.VMEM),
        scratch_shapes=[
            pltpu.VMEM((HB, 2 * D), jnp.float32),
            pltpu.VMEM((HB, 2 * D), jnp.float32),
            pltpu.SemaphoreType.DMA,
            pltpu.SemaphoreType.DMA,
        ],
        compiler_params=pltpu.CompilerParams(collective_id=0),
    )(Q, K.reshape(n_keys, HD), V.reshape(n_keys, HD), bt, lens)
